# baseline (speedup 1.0000x reference)
import functools
import math

import jax
import jax.numpy as jnp
from jax import lax
from jax.experimental import pallas as pl
from jax.experimental.pallas import tpu as pltpu
from jax.experimental.pallas import tpu_sc as plsc

D_MODEL = 64
SCALE = math.sqrt(D_MODEL)

NC = 2
NS = 16
NW = NC * NS
LANES = 16

CHUNK_X = 4
NBUF = 2
KD = 192

_MESH = dict(core_axis_name="c", subcore_axis_name="s",
             num_cores=NC, num_subcores=NS)


@jax.jit
def _depad_table(table):
    V = table.shape[0]
    n_slabs = V * D_MODEL // 1024
    sl_base = n_slabs // NW
    sl_extra = n_slabs % NW
    full_d = (sl_base * 16) // KD
    tail_lo = sl_base * 16 - full_d * KD

    mesh = plsc.VectorSubcoreMesh(**_MESH)

    @functools.partial(
        pl.kernel,
        mesh=mesh,
        compiler_params=pltpu.CompilerParams(use_tc_tiling_on_sc=True),
        out_type=jax.ShapeDtypeStruct((n_slabs, 8, 128), jnp.float32),
        scratch_types=[
            pltpu.VMEM((NBUF, KD, D_MODEL), jnp.float32),
            pltpu.VMEM((NBUF, KD // 16, 8, 128), jnp.float32),
            pltpu.SemaphoreType.DMA,
            pltpu.SemaphoreType.DMA,
        ],
    )
    def body(table_hbm, out_hbm, rbuf, pbuf, wsem0, wsem1):
        wsems = (wsem0, wsem1)
        wid = lax.axis_index("s") * NC + lax.axis_index("c")
        slab0 = wid * sl_base + jnp.minimum(wid, sl_extra)
        row0 = slab0 * 16

        def repack(b, nrows):
            def row_body(r, carry):
                a = r // 16
                b2 = (r % 16) // 2
                oo = (r % 2) * D_MODEL
                for jj in range(D_MODEL // LANES):
                    pbuf[b, a, b2, pl.ds(oo + jj * LANES, LANES)] = (
                        rbuf[b, r, pl.ds(jj * LANES, LANES)])
                return carry
            lax.fori_loop(0, nrows, row_body, 0, unroll=4)

        def wait_w(b, nslabs):
            pltpu.make_async_copy(
                pbuf.at[b, pl.ds(0, nslabs)], out_hbm.at[pl.ds(0, nslabs)],
                wsems[b]).wait()

        def group(i, carry):
            for b in range(NBUF):
                c = i * NBUF + b
                r = row0 + c * KD

                @pl.when(c >= NBUF)
                def _():
                    wait_w(b, KD // 16)
                pltpu.sync_copy(table_hbm.at[pl.ds(r, KD)], rbuf.at[b])
                repack(b, KD)
                pltpu.async_copy(
                    pbuf.at[b],
                    out_hbm.at[pl.ds(slab0 + c * (KD // 16), KD // 16)],
                    wsems[b])
            return carry

        lax.fori_loop(0, full_d // NBUF, group, 0)
        wait_w(0, KD // 16)
        wait_w(1, KD // 16)

        tail_r = row0 + full_d * KD
        tail_s = slab0 + full_d * (KD // 16)
        pltpu.sync_copy(
            table_hbm.at[pl.ds(tail_r, tail_lo)],
            rbuf.at[0, pl.ds(0, tail_lo)])
        repack(0, tail_lo)
        pltpu.async_copy(
            pbuf.at[0, pl.ds(0, tail_lo // 16)],
            out_hbm.at[pl.ds(tail_s, tail_lo // 16)], wsems[0])
        wait_w(0, tail_lo // 16)

        @pl.when(wid < sl_extra)
        def _():
            pltpu.sync_copy(
                table_hbm.at[pl.ds(tail_r + tail_lo, 16)],
                rbuf.at[1, pl.ds(0, 16)])
            repack(1, 16)
            pltpu.async_copy(
                pbuf.at[1, pl.ds(0, 1)],
                out_hbm.at[pl.ds(tail_s + tail_lo // 16, 1)], wsems[1])
            wait_w(1, 1)

    return body(table)


@functools.partial(jax.jit, static_argnames=("seq",))
def _emb_lookup(idx, table, seq):
    n_x = idx.shape[0]
    x_per_w = n_x // NW
    n_chunks = x_per_w // CHUNK_X
    chunk_rows = CHUNK_X * seq
    rows_per_w = x_per_w * seq
    n_rows = n_x * seq
    splits = []
    off = 0
    while off < seq:
        w = min(128, seq - off)
        splits.append((off, w))
        off += w

    mesh = plsc.VectorSubcoreMesh(**_MESH)

    @functools.partial(
        pl.kernel,
        mesh=mesh,
        compiler_params=pltpu.CompilerParams(use_tc_tiling_on_sc=False),
        out_type=jax.ShapeDtypeStruct((n_rows, D_MODEL), jnp.float32),
        scratch_types=[
            pltpu.VMEM((NBUF, CHUNK_X, seq), jnp.int32),
            pltpu.VMEM((NBUF, chunk_rows, D_MODEL), jnp.float32),
            pltpu.SemaphoreType.DMA,
            pltpu.SemaphoreType.DMA,
            pltpu.SemaphoreType.DMA,
            pltpu.SemaphoreType.DMA,
        ],
    )
    def body(idx_hbm, table_hbm, out_hbm, idx_v, rows_v,
             gsem0, gsem1, ssem0, ssem1):
        gsems = (gsem0, gsem1)
        ssems = (ssem0, ssem1)
        wid = lax.axis_index("s") * NC + lax.axis_index("c")
        xrow0 = wid * x_per_w
        base = wid * rows_per_w

        def start_gather(c, b):
            pltpu.sync_copy(
                idx_hbm.at[pl.ds(xrow0 + c * CHUNK_X, CHUNK_X)],
                idx_v.at[b])
            for r in range(CHUNK_X):
                for (o, w) in splits:
                    pltpu.async_copy(
                        table_hbm.at[idx_v.at[b, r, pl.ds(o, w)]],
                        rows_v.at[b, pl.ds(r * seq + o, w)],
                        gsems[b])

        def wait_gather(b):
            pltpu.make_async_copy(
                table_hbm.at[pl.ds(0, chunk_rows)], rows_v.at[b],
                gsems[b]).wait()

        def scale_chunk(b):
            def row_body(r, carry):
                for jj in range(D_MODEL // LANES):
                    sl = pl.ds(jj * LANES, LANES)
                    rows_v[b, r, sl] = rows_v[b, r, sl] * SCALE
                return carry
            lax.fori_loop(0, chunk_rows, row_body, 0, unroll=4)

        def start_store(c, b):
            pltpu.async_copy(
                rows_v.at[b],
                out_hbm.at[pl.ds(base + c * chunk_rows, chunk_rows)],
                ssems[b])

        def wait_store(b):
            pltpu.make_async_copy(
                rows_v.at[b], out_hbm.at[pl.ds(0, chunk_rows)],
                ssems[b]).wait()

        start_gather(0, 0)

        def group_body(i, carry):
            g = i * NBUF
            for b in range(NBUF):
                c = g + b
                nb = (b + 1) % NBUF
                nxt = c + 1

                @pl.when(nxt < n_chunks)
                def _prefetch():
                    @pl.when(nxt >= NBUF)
                    def _reclaim():
                        wait_store(nb)
                    start_gather(nxt, nb)

                wait_gather(b)
                scale_chunk(b)
                start_store(c, b)
            return carry

        lax.fori_loop(0, n_chunks // NBUF, group_body, 0)
        for b in range(NBUF):
            wait_store(b)

    return body(idx, table)


def kernel(X, table):
    idx = X.astype(jnp.int32)
    packed = _depad_table(table).reshape(table.shape[0], D_MODEL)
    out = _emb_lookup(idx, packed, X.shape[1])
    return out.reshape(X.shape[0], X.shape[1], D_MODEL)


# speedup vs baseline: 1.4083x; 1.4083x over previous
import functools
import math

import jax
import jax.numpy as jnp
from jax import lax
from jax.experimental import pallas as pl
from jax.experimental.pallas import tpu as pltpu
from jax.experimental.pallas import tpu_sc as plsc

D_MODEL = 64
SCALE = math.sqrt(D_MODEL)

NC = 2
NS = 16
NW = NC * NS
LANES = 16

CHUNK_X = 4
NBUF = 2


@functools.partial(jax.jit, static_argnames=("seq",))
def _emb_lookup(idx, table, seq):
    n_x = idx.shape[0]
    x_per_w = n_x // NW
    n_chunks = x_per_w // CHUNK_X
    chunk_rows = CHUNK_X * seq
    rows_per_w = x_per_w * seq
    n_rows = n_x * seq
    splits = []
    off = 0
    while off < seq:
        w = min(128, seq - off)
        splits.append((off, w))
        off += w

    mesh = plsc.VectorSubcoreMesh(
        core_axis_name="c", subcore_axis_name="s",
        num_cores=NC, num_subcores=NS)

    @functools.partial(
        pl.kernel,
        mesh=mesh,
        compiler_params=pltpu.CompilerParams(use_tc_tiling_on_sc=False),
        out_type=jax.ShapeDtypeStruct((n_rows, D_MODEL), jnp.float32),
        scratch_types=[
            pltpu.VMEM((NBUF, CHUNK_X, seq), jnp.int32),
            pltpu.VMEM((NBUF, chunk_rows, D_MODEL), jnp.float32),
            pltpu.SemaphoreType.DMA,
            pltpu.SemaphoreType.DMA,
            pltpu.SemaphoreType.DMA,
            pltpu.SemaphoreType.DMA,
        ],
    )
    def body(idx_hbm, table_hbm, out_hbm, idx_v, rows_v,
             gsem0, gsem1, ssem0, ssem1):
        gsems = (gsem0, gsem1)
        ssems = (ssem0, ssem1)
        wid = lax.axis_index("s") * NC + lax.axis_index("c")
        xrow0 = wid * x_per_w
        base = wid * rows_per_w

        def start_gather(c, b):
            pltpu.sync_copy(
                idx_hbm.at[pl.ds(xrow0 + c * CHUNK_X, CHUNK_X)],
                idx_v.at[b])
            for r in range(CHUNK_X):
                for (o, w) in splits:
                    pltpu.async_copy(
                        table_hbm.at[idx_v.at[b, r, pl.ds(o, w)]],
                        rows_v.at[b, pl.ds(r * seq + o, w)],
                        gsems[b])

        def wait_gather(b):
            pltpu.make_async_copy(
                table_hbm.at[pl.ds(0, chunk_rows)], rows_v.at[b],
                gsems[b]).wait()

        def scale_chunk(b):
            def row_body(r, carry):
                for jj in range(D_MODEL // LANES):
                    sl = pl.ds(jj * LANES, LANES)
                    rows_v[b, r, sl] = rows_v[b, r, sl] * SCALE
                return carry
            lax.fori_loop(0, chunk_rows, row_body, 0, unroll=4)

        def start_store(c, b):
            pltpu.async_copy(
                rows_v.at[b],
                out_hbm.at[pl.ds(base + c * chunk_rows, chunk_rows)],
                ssems[b])

        def wait_store(b):
            pltpu.make_async_copy(
                rows_v.at[b], out_hbm.at[pl.ds(0, chunk_rows)],
                ssems[b]).wait()

        start_gather(0, 0)

        def group_body(i, carry):
            g = i * NBUF
            for b in range(NBUF):
                c = g + b
                nb = (b + 1) % NBUF
                nxt = c + 1

                @pl.when(nxt < n_chunks)
                def _prefetch():
                    @pl.when(nxt >= NBUF)
                    def _reclaim():
                        wait_store(nb)
                    start_gather(nxt, nb)

                wait_gather(b)
                scale_chunk(b)
                start_store(c, b)
            return carry

        lax.fori_loop(0, n_chunks // NBUF, group_body, 0)
        for b in range(NBUF):
            wait_store(b)

    return body(idx, table)


def kernel(X, table):
    idx = X.astype(jnp.int32)
    out = _emb_lookup(idx, table, X.shape[1])
    return out.reshape(X.shape[0], X.shape[1], D_MODEL)
